# Initial kernel scaffold; baseline (speedup 1.0000x reference)
import jax
import jax.numpy as jnp
from jax.experimental import pallas as pl


def kernel(x, wte, wpe):
    raise NotImplementedError("write your pallas kernel here")


# speedup vs baseline: 1.0194x; 1.0194x over previous
import functools

import jax
import jax.numpy as jnp
from jax import lax
from jax.experimental import pallas as pl
from jax.experimental.pallas import tpu as pltpu
from jax.experimental.pallas import tpu_sc as plsc

_B, _S, _D = 4, 2048, 768
_NC, _NS = 2, 16
_NW = _NC * _NS
_PPW = _S // _NW
_CH = 32
_NCHUNK = _B * (_PPW // _CH)
_LPR = _D // 16

_mesh = plsc.VectorSubcoreMesh(core_axis_name="c", subcore_axis_name="s")


@functools.partial(
    pl.kernel,
    mesh=_mesh,
    out_type=jax.ShapeDtypeStruct((_B, _NW, _PPW, _D), jnp.float32),
    scratch_types=[
        pltpu.VMEM((_B, _PPW), jnp.int32),
        pltpu.VMEM((_PPW, _D), jnp.float32),
        pltpu.VMEM((3, _CH, _D), jnp.float32),
        pltpu.SemaphoreType.DMA,
        pltpu.SemaphoreType.DMA,
        pltpu.SemaphoreType.DMA,
        pltpu.SemaphoreType.DMA,
        pltpu.SemaphoreType.DMA,
        pltpu.SemaphoreType.DMA,
        pltpu.SemaphoreType.DMA,
        pltpu.SemaphoreType.DMA,
    ],
)
def _emb_kernel(x_hbm, wte_hbm, wpe_hbm, out_hbm,
                idx_v, wpe_v, tok_v,
                sem_idx, sem_wpe, g0, g1, g2, o0, o1, o2):
    gsem = (g0, g1, g2)
    osem = (o0, o1, o2)
    wid = lax.axis_index("s") * _NC + lax.axis_index("c")

    idx_copies = [
        pltpu.async_copy(x_hbm.at[b, wid], idx_v.at[b], sem_idx)
        for b in range(_B)
    ]
    wpe_copy = pltpu.async_copy(
        wpe_hbm.at[pl.ds(wid * _PPW, _PPW)], wpe_v, sem_wpe)
    for c in idx_copies:
        c.wait()

    def gather_start(c, bi):
        b, half = c // 2, c % 2
        return pltpu.async_copy(
            wte_hbm.at[idx_v.at[b, pl.ds(half * _CH, _CH)]],
            tok_v.at[bi], gsem[bi])

    def store_start(c, bi):
        b, half = c // 2, c % 2
        return pltpu.async_copy(
            tok_v.at[bi],
            out_hbm.at[b, wid, pl.ds(half * _CH, _CH)], osem[bi])

    def add_chunk(bi, half):
        def row_body(r, carry):
            for k in range(_LPR):
                sl = pl.ds(k * 16, 16)
                plsc.addupdate(tok_v.at[bi, r, sl],
                               wpe_v[half * _CH + r, sl])
            return carry
        lax.fori_loop(0, _CH, row_body, 0)

    hg = [None] * _NCHUNK
    ho = [None] * _NCHUNK
    hg[0] = gather_start(0, 0)
    hg[1] = gather_start(1, 1)
    wpe_copy.wait()
    for c in range(_NCHUNK):
        bi = c % 3
        if c + 2 < _NCHUNK:
            if c >= 1:
                ho[c - 1].wait()
            hg[c + 2] = gather_start(c + 2, (c + 2) % 3)
        hg[c].wait()
        add_chunk(bi, c % 2)
        ho[c] = store_start(c, bi)
    for c in range(_NCHUNK - 3, _NCHUNK):
        ho[c].wait()


def kernel(x, wte, wpe):
    xr = x.astype(jnp.int32).reshape(_B, _NW, _PPW)
    out = _emb_kernel(xr, wte, wpe)
    return out.reshape(_B, _S, _D)


# speedup vs baseline: 1.3057x; 1.2809x over previous
import functools

import jax
import jax.numpy as jnp
from jax import lax
from jax.experimental import pallas as pl
from jax.experimental.pallas import tpu as pltpu
from jax.experimental.pallas import tpu_sc as plsc

_B, _S, _D = 4, 2048, 768
_NC, _NS = 2, 16
_NW = _NC * _NS
_PPW = _S // _NW
_CH = 32
_NCHUNK = _B * (_PPW // _CH)
_LPR = _D // 16

_mesh = plsc.VectorSubcoreMesh(core_axis_name="c", subcore_axis_name="s")


@functools.partial(
    pl.kernel,
    mesh=_mesh,
    out_type=jax.ShapeDtypeStruct((_B, _NW, _PPW, _D), jnp.float32),
    scratch_types=[
        pltpu.VMEM((_B, _PPW), jnp.int32),
        pltpu.VMEM((_PPW, _D), jnp.float32),
        pltpu.VMEM((3, _CH, _D), jnp.float32),
        pltpu.SemaphoreType.DMA,
        pltpu.SemaphoreType.DMA,
        pltpu.SemaphoreType.DMA,
        pltpu.SemaphoreType.DMA,
        pltpu.SemaphoreType.DMA,
        pltpu.SemaphoreType.DMA,
        pltpu.SemaphoreType.DMA,
        pltpu.SemaphoreType.DMA,
    ],
)
def _emb_kernel(x_hbm, wte_hbm, wpe_hbm, out_hbm,
                idx_v, wpe_v, tok_v,
                sem_idx, sem_wpe, g0, g1, g2, o0, o1, o2):
    gsem = (g0, g1, g2)
    osem = (o0, o1, o2)
    wid = lax.axis_index("s") * _NC + lax.axis_index("c")

    idx_copies = [
        pltpu.async_copy(x_hbm.at[b, wid], idx_v.at[b], sem_idx)
        for b in range(_B)
    ]
    wpe_copy = pltpu.async_copy(
        wpe_hbm.at[pl.ds(wid * _PPW, _PPW)], wpe_v, sem_wpe)
    for c in idx_copies:
        c.wait()

    def gather_start(c, bi):
        b, half = c // 2, c % 2
        return pltpu.async_copy(
            wte_hbm.at[idx_v.at[b, pl.ds(half * _CH, _CH)]],
            tok_v.at[bi], gsem[bi])

    def store_start(c, bi):
        b, half = c // 2, c % 2
        return pltpu.async_copy(
            tok_v.at[bi],
            out_hbm.at[b, wid, pl.ds(half * _CH, _CH)], osem[bi])

    def add_chunk(bi, half):
        @plsc.parallel_loop(0, _CH, unroll=2)
        def row_body(r):
            for g in range(_LPR // 8):
                w = [wpe_v[half * _CH + r, pl.ds((g * 8 + j) * 16, 16)]
                     for j in range(8)]
                for j in range(8):
                    plsc.addupdate(
                        tok_v.at[bi, r, pl.ds((g * 8 + j) * 16, 16)], w[j])

    hg = [None] * _NCHUNK
    ho = [None] * _NCHUNK
    hg[0] = gather_start(0, 0)
    hg[1] = gather_start(1, 1)
    wpe_copy.wait()
    for c in range(_NCHUNK):
        bi = c % 3
        if c + 2 < _NCHUNK:
            if c >= 1:
                ho[c - 1].wait()
            hg[c + 2] = gather_start(c + 2, (c + 2) % 3)
        hg[c].wait()
        add_chunk(bi, c % 2)
        ho[c] = store_start(c, bi)
    for c in range(_NCHUNK - 3, _NCHUNK):
        ho[c].wait()


def kernel(x, wte, wpe):
    xr = x.astype(jnp.int32).reshape(_B, _NW, _PPW)
    out = _emb_kernel(xr, wte, wpe)
    return out.reshape(_B, _S, _D)


# speedup vs baseline: 1.3080x; 1.0017x over previous
import functools

import jax
import jax.numpy as jnp
from jax import lax
from jax.experimental import pallas as pl
from jax.experimental.pallas import tpu as pltpu
from jax.experimental.pallas import tpu_sc as plsc

_B, _S, _D = 4, 2048, 768
_NC, _NS = 2, 16
_NW = _NC * _NS
_PPW = _S // _NW
_CH = 32
_NCHUNK = _B * (_PPW // _CH)
_LPR = _D // 16

_mesh = plsc.VectorSubcoreMesh(core_axis_name="c", subcore_axis_name="s")


@functools.partial(
    pl.kernel,
    mesh=_mesh,
    out_type=jax.ShapeDtypeStruct((_B, _S, _D), jnp.float32),
    scratch_types=[
        pltpu.VMEM((_B, _PPW), jnp.int32),
        pltpu.VMEM((_PPW, _D), jnp.float32),
        pltpu.VMEM((3, _CH, _D), jnp.float32),
        pltpu.SemaphoreType.DMA,
        pltpu.SemaphoreType.DMA,
        pltpu.SemaphoreType.DMA,
        pltpu.SemaphoreType.DMA,
        pltpu.SemaphoreType.DMA,
        pltpu.SemaphoreType.DMA,
        pltpu.SemaphoreType.DMA,
        pltpu.SemaphoreType.DMA,
    ],
)
def _emb_kernel(x_hbm, wte_hbm, wpe_hbm, out_hbm,
                idx_v, wpe_v, tok_v,
                sem_idx, sem_wpe, g0, g1, g2, o0, o1, o2):
    gsem = (g0, g1, g2)
    osem = (o0, o1, o2)
    wid = lax.axis_index("s") * _NC + lax.axis_index("c")
    pos0 = wid * _PPW

    idx_copies = [
        pltpu.async_copy(x_hbm.at[b, pl.ds(pos0, _PPW)], idx_v.at[b], sem_idx)
        for b in range(_B)
    ]
    wpe_copy = pltpu.async_copy(wpe_hbm.at[pl.ds(pos0, _PPW)], wpe_v, sem_wpe)
    for c in idx_copies:
        c.wait()

    def gather_start(c, bi):
        b, half = c // 2, c % 2
        return pltpu.async_copy(
            wte_hbm.at[idx_v.at[b, pl.ds(half * _CH, _CH)]],
            tok_v.at[bi], gsem[bi])

    def store_start(c, bi):
        b, half = c // 2, c % 2
        return pltpu.async_copy(
            tok_v.at[bi],
            out_hbm.at[b, pl.ds(pos0 + half * _CH, _CH)], osem[bi])

    def add_chunk(bi, half):
        @plsc.parallel_loop(0, _CH, unroll=2)
        def row_body(r):
            for g in range(_LPR // 8):
                w = [wpe_v[half * _CH + r, pl.ds((g * 8 + j) * 16, 16)]
                     for j in range(8)]
                for j in range(8):
                    plsc.addupdate(
                        tok_v.at[bi, r, pl.ds((g * 8 + j) * 16, 16)], w[j])

    hg = [None] * _NCHUNK
    ho = [None] * _NCHUNK
    hg[0] = gather_start(0, 0)
    hg[1] = gather_start(1, 1)
    wpe_copy.wait()
    for c in range(_NCHUNK):
        bi = c % 3
        if c + 2 < _NCHUNK:
            if c >= 1:
                ho[c - 1].wait()
            hg[c + 2] = gather_start(c + 2, (c + 2) % 3)
        hg[c].wait()
        add_chunk(bi, c % 2)
        ho[c] = store_start(c, bi)
    for c in range(_NCHUNK - 3, _NCHUNK):
        ho[c].wait()


def kernel(x, wte, wpe):
    return _emb_kernel(x.astype(jnp.int32), wte, wpe)
